# Initial kernel scaffold; baseline (speedup 1.0000x reference)
import jax
import jax.numpy as jnp
from jax.experimental import pallas as pl


def kernel(pixel_values, W_patch, b_patch, cls_token, pos_embed):
    raise NotImplementedError("write your pallas kernel here")


# speedup vs baseline: 1.6318x; 1.6318x over previous
import jax
import jax.numpy as jnp
from jax.experimental import pallas as pl

B = 64
IMG = 384
P = 16
C = 3
HID = 384
HP = IMG // P
N = HP * HP
TOPK = 288
F = C * P * P


def _pretokenize(pixel_values):
    x = pixel_values.reshape(B, C, HP, P, HP, P)
    x = x.transpose(0, 2, 4, 1, 3, 5)
    return x.reshape(B, N, F)


def _body(x_ref, sr_ref, sc_ref, w_ref, b_ref, posr_ref, cls_ref, o_ref):
    x = x_ref[0]
    s_row = sr_ref[0]
    s_col = sc_ref[0]

    ii = jax.lax.broadcasted_iota(jnp.int32, (N, N), 0)
    jj = jax.lax.broadcasted_iota(jnp.int32, (N, N), 1)
    beats = (s_col > s_row) | ((s_col == s_row) & (ii < jj))
    bf = beats.astype(jnp.float32)
    wins_col = jnp.sum(bf, axis=1, keepdims=True)
    sel_col = wins_col >= jnp.float32(N - TOPK)
    rank_row = jnp.sum(bf, axis=0, keepdims=True)
    sel_row = rank_row < jnp.float32(TOPK)

    lower = ((ii < jj) & sel_col).astype(jnp.float32)
    pos_row = jnp.sum(lower, axis=0, keepdims=True)

    kk = jax.lax.broadcasted_iota(jnp.int32, (TOPK, N), 0).astype(jnp.float32)
    S = ((pos_row == kk) & sel_row).astype(jnp.float32)

    emb = jnp.dot(x, w_ref[...], preferred_element_type=jnp.float32)
    emb = emb + b_ref[...] + posr_ref[...]

    g = jax.lax.dot(S, emb, precision=jax.lax.Precision.HIGHEST,
                    preferred_element_type=jnp.float32)

    o_ref[0, 0:1, :] = cls_ref[...]
    o_ref[0, 1:, :] = g


def kernel(pixel_values, W_patch, b_patch, cls_token, pos_embed):
    patches = _pretokenize(pixel_values)
    scores = jnp.var(patches, axis=-1, ddof=1)
    s_row = scores.reshape(B, 1, N)
    s_col = scores.reshape(B, N, 1)
    b_row = b_patch.reshape(1, HID)
    pos_rest = pos_embed[0, 1:, :]
    cls_row = (cls_token[0] + pos_embed[0, :1, :]).reshape(1, HID)

    out = pl.pallas_call(
        _body,
        grid=(B,),
        in_specs=[
            pl.BlockSpec((1, N, F), lambda b: (b, 0, 0)),
            pl.BlockSpec((1, 1, N), lambda b: (b, 0, 0)),
            pl.BlockSpec((1, N, 1), lambda b: (b, 0, 0)),
            pl.BlockSpec((F, HID), lambda b: (0, 0)),
            pl.BlockSpec((1, HID), lambda b: (0, 0)),
            pl.BlockSpec((N, HID), lambda b: (0, 0)),
            pl.BlockSpec((1, HID), lambda b: (0, 0)),
        ],
        out_specs=pl.BlockSpec((1, TOPK + 1, HID), lambda b: (b, 0, 0)),
        out_shape=jax.ShapeDtypeStruct((B, TOPK + 1, HID), jnp.float32),
    )(patches, s_row, s_col, W_patch, b_row, pos_rest, cls_row)
    return out


# speedup vs baseline: 1.7703x; 1.0849x over previous
import jax
import jax.numpy as jnp
from jax.experimental import pallas as pl

B = 64
IMG = 384
P = 16
C = 3
HID = 384
HP = IMG // P
N = HP * HP
TOPK = 288
F = C * P * P


def _pretokenize(pixel_values):
    x = pixel_values.reshape(B, C, HP, P, HP, P)
    x = x.transpose(0, 2, 4, 1, 3, 5)
    return x.reshape(B, N, F)


def _body(x_ref, sr_ref, sc_ref, w_ref, b_ref, posr_ref, cls_ref, o_ref):
    x = x_ref[0]
    s_row = sr_ref[0]
    s_col = sc_ref[0]

    ii = jax.lax.broadcasted_iota(jnp.int32, (N, N), 0)
    jj = jax.lax.broadcasted_iota(jnp.int32, (N, N), 1)
    beats = (s_col > s_row) | ((s_col == s_row) & (ii < jj))
    bf = beats.astype(jnp.float32)
    wins_col = jnp.sum(bf, axis=1, keepdims=True)
    sel_col = wins_col >= jnp.float32(N - TOPK)
    rank_row = jnp.sum(bf, axis=0, keepdims=True)
    sel_row = rank_row < jnp.float32(TOPK)

    lower = ((ii < jj) & sel_col).astype(jnp.float32)
    pos_row = jnp.sum(lower, axis=0, keepdims=True)

    kk = jax.lax.broadcasted_iota(jnp.int32, (TOPK, N), 0).astype(jnp.float32)
    S = ((pos_row == kk) & sel_row).astype(jnp.float32)

    emb = jnp.dot(x, w_ref[...], preferred_element_type=jnp.float32)
    emb = emb + b_ref[...] + posr_ref[...]

    g = jnp.dot(S, emb, preferred_element_type=jnp.float32)

    o_ref[0, 0:1, :] = cls_ref[...]
    o_ref[0, 1:, :] = g


def kernel(pixel_values, W_patch, b_patch, cls_token, pos_embed):
    patches = _pretokenize(pixel_values)
    scores = jnp.var(patches, axis=-1, ddof=1)
    s_row = scores.reshape(B, 1, N)
    s_col = scores.reshape(B, N, 1)
    b_row = b_patch.reshape(1, HID)
    pos_rest = pos_embed[0, 1:, :]
    cls_row = (cls_token[0] + pos_embed[0, :1, :]).reshape(1, HID)

    out = pl.pallas_call(
        _body,
        grid=(B,),
        in_specs=[
            pl.BlockSpec((1, N, F), lambda b: (b, 0, 0)),
            pl.BlockSpec((1, 1, N), lambda b: (b, 0, 0)),
            pl.BlockSpec((1, N, 1), lambda b: (b, 0, 0)),
            pl.BlockSpec((F, HID), lambda b: (0, 0)),
            pl.BlockSpec((1, HID), lambda b: (0, 0)),
            pl.BlockSpec((N, HID), lambda b: (0, 0)),
            pl.BlockSpec((1, HID), lambda b: (0, 0)),
        ],
        out_specs=pl.BlockSpec((1, TOPK + 1, HID), lambda b: (b, 0, 0)),
        out_shape=jax.ShapeDtypeStruct((B, TOPK + 1, HID), jnp.float32),
    )(patches, s_row, s_col, W_patch, b_row, pos_rest, cls_row)
    return out
